# baseline (speedup 1.0000x reference)
import jax
import jax.numpy as jnp
from jax.experimental import pallas as pl
from jax.experimental.pallas import tpu as pltpu


def _make_body(E, NB, BLK):
    def _moe_body(x_ref, wr_ref, wg_ref, wu_ref, wd_ref, out_ref,
                  comb_ref, gacc_ref, uacc_ref, h0_ref, h1_ref):
        e = pl.program_id(0)
        i = pl.program_id(1)
        T, _ = comb_ref.shape

        @pl.when((e == 0) & (i == 0))
        def _route():
            x = x_ref[...]
            logits = jnp.dot(x, wr_ref[...], preferred_element_type=jnp.float32)
            aff = jax.nn.softmax(logits, axis=-1)
            idx = jax.lax.broadcasted_iota(jnp.int32, (T, E), 1)
            m1 = jnp.max(aff, axis=-1, keepdims=True)
            i1 = jnp.min(jnp.where(aff == m1, idx, E), axis=-1, keepdims=True)
            mask1 = idx == i1
            aff2 = jnp.where(mask1, -1.0, aff)
            m2 = jnp.max(aff2, axis=-1, keepdims=True)
            i2 = jnp.min(jnp.where(aff2 == m2, idx, E), axis=-1, keepdims=True)
            mask2 = idx == i2
            denom = m1 + m2
            comb_ref[...] = (
                jnp.where(mask1, m1 / denom, 0.0)
                + jnp.where(mask2, m2 / denom, 0.0)
            )

        @pl.when(e < E)
        def _gate_up():
            xs = jax.lax.switch(
                i, [lambda k=k: x_ref[:, k * BLK:(k + 1) * BLK] for k in range(NB)]
            )
            gpart = jnp.dot(xs, wg_ref[0], preferred_element_type=jnp.float32)
            upart = jnp.dot(xs, wu_ref[0], preferred_element_type=jnp.float32)

            @pl.when(i == 0)
            def _():
                gacc_ref[...] = gpart
                uacc_ref[...] = upart

            @pl.when(i > 0)
            def _():
                gacc_ref[...] += gpart
                uacc_ref[...] += upart

            @pl.when(i == NB - 1)
            def _():
                g = gacc_ref[...]
                h = g * jax.nn.sigmoid(g) * uacc_ref[...]
                comb = comb_ref[...]
                idx = jax.lax.broadcasted_iota(jnp.int32, (T, E), 1)
                w_e = jnp.sum(jnp.where(idx == e, comb, 0.0), axis=-1,
                              keepdims=True)
                h = h * w_e

                @pl.when(e % 2 == 0)
                def _():
                    h0_ref[...] = h

                @pl.when(e % 2 == 1)
                def _():
                    h1_ref[...] = h

        @pl.when(e > 0)
        def _down():
            sel = (e - 1) % 2

            def mk(ref, k):
                return lambda: ref[:, k * BLK:(k + 1) * BLK]

            branches = [mk(h0_ref, k) for k in range(NB)]
            branches += [mk(h1_ref, k) for k in range(NB)]
            h = jax.lax.switch(i + NB * sel, branches)
            partial = jnp.dot(h, wd_ref[0], preferred_element_type=jnp.float32)

            @pl.when((e == 1) & (i == 0))
            def _():
                out_ref[...] = partial

            @pl.when((e > 1) | (i > 0))
            def _():
                out_ref[...] += partial

    return _moe_body


def kernel(x, W_router, W_gate, W_up, W_down):
    B, S, H = x.shape
    E, _, I = W_gate.shape
    T = B * S
    BLK = 512
    NB = H // BLK

    def gu_map(e, i):
        ec = jnp.minimum(e, E - 1)
        ic = jnp.where(e == E, NB - 1, i)
        return (ec, ic, 0)

    def d_map(e, i):
        ec = jnp.maximum(e, 1) - 1
        ic = jnp.where(e == 0, 0, i)
        return (ec, ic, 0)

    out = pl.pallas_call(
        _make_body(E, NB, BLK),
        grid=(E + 1, NB),
        in_specs=[
            pl.BlockSpec((T, None, H), lambda e, i: (0, 0, 0)),
            pl.BlockSpec((H, E), lambda e, i: (0, 0)),
            pl.BlockSpec((1, BLK, I), gu_map),
            pl.BlockSpec((1, BLK, I), gu_map),
            pl.BlockSpec((1, BLK, H), d_map),
        ],
        out_specs=pl.BlockSpec((T, None, H), lambda e, i: (0, 0, 0)),
        out_shape=jax.ShapeDtypeStruct((B, S, H), x.dtype),
        scratch_shapes=[
            pltpu.VMEM((T, E), jnp.float32),
            pltpu.VMEM((T, I), jnp.float32),
            pltpu.VMEM((T, I), jnp.float32),
            pltpu.VMEM((T, I), jnp.float32),
            pltpu.VMEM((T, I), jnp.float32),
        ],
        compiler_params=pltpu.CompilerParams(
            dimension_semantics=("arbitrary", "arbitrary"),
        ),
    )(x, W_router, W_gate, W_up, W_down)
    return out


# speedup vs baseline: 1.0153x; 1.0153x over previous
import jax
import jax.numpy as jnp
from jax.experimental import pallas as pl
from jax.experimental.pallas import tpu as pltpu


def _moe_body(x_ref, wr_ref, wga_ref, wgb_ref, wua_ref, wub_ref,
              wda_ref, wdb_ref, out_ref, comb_ref):
    e = pl.program_id(0)
    ib = pl.program_id(1)
    T, E = comb_ref.shape

    @pl.when((e == 0) & (ib == 0))
    def _init():
        x = x_ref[...]
        logits = jnp.dot(x, wr_ref[...], preferred_element_type=jnp.float32)
        aff = jax.nn.softmax(logits, axis=-1)
        idx = jax.lax.broadcasted_iota(jnp.int32, (T, E), 1)
        m1 = jnp.max(aff, axis=-1, keepdims=True)
        i1 = jnp.min(jnp.where(aff == m1, idx, E), axis=-1, keepdims=True)
        mask1 = idx == i1
        aff2 = jnp.where(mask1, -1.0, aff)
        m2 = jnp.max(aff2, axis=-1, keepdims=True)
        i2 = jnp.min(jnp.where(aff2 == m2, idx, E), axis=-1, keepdims=True)
        mask2 = idx == i2
        denom = m1 + m2
        comb_ref[...] = (
            jnp.where(mask1, m1 / denom, 0.0) + jnp.where(mask2, m2 / denom, 0.0)
        )
        out_ref[...] = jnp.zeros_like(out_ref)

    x = x_ref[...]
    comb = comb_ref[...]
    idx = jax.lax.broadcasted_iota(jnp.int32, (T, E), 1)
    w_e = jnp.sum(jnp.where(idx == e, comb, 0.0), axis=-1, keepdims=True)

    gate_a = jnp.dot(x, wga_ref[0], preferred_element_type=jnp.float32)
    up_a = jnp.dot(x, wua_ref[0], preferred_element_type=jnp.float32)
    hmid_a = gate_a * jax.nn.sigmoid(gate_a) * up_a * w_e
    acc = jnp.dot(hmid_a, wda_ref[0], preferred_element_type=jnp.float32)

    gate_b = jnp.dot(x, wgb_ref[0], preferred_element_type=jnp.float32)
    up_b = jnp.dot(x, wub_ref[0], preferred_element_type=jnp.float32)
    hmid_b = gate_b * jax.nn.sigmoid(gate_b) * up_b * w_e
    acc += jnp.dot(hmid_b, wdb_ref[0], preferred_element_type=jnp.float32)

    out_ref[...] += acc


def kernel(x, W_router, W_gate, W_up, W_down):
    B, S, H = x.shape
    E, _, I = W_gate.shape
    T = B * S
    HB = 256
    n_ib = I // (2 * HB)

    out = pl.pallas_call(
        _moe_body,
        grid=(E, n_ib),
        in_specs=[
            pl.BlockSpec((T, None, H), lambda e, i: (0, 0, 0)),
            pl.BlockSpec((H, E), lambda e, i: (0, 0)),
            pl.BlockSpec((1, H, HB), lambda e, i: (e, 0, 2 * i)),
            pl.BlockSpec((1, H, HB), lambda e, i: (e, 0, 2 * i + 1)),
            pl.BlockSpec((1, H, HB), lambda e, i: (e, 0, 2 * i)),
            pl.BlockSpec((1, H, HB), lambda e, i: (e, 0, 2 * i + 1)),
            pl.BlockSpec((1, HB, H), lambda e, i: (e, 2 * i, 0)),
            pl.BlockSpec((1, HB, H), lambda e, i: (e, 2 * i + 1, 0)),
        ],
        out_specs=pl.BlockSpec((T, None, H), lambda e, i: (0, 0, 0)),
        out_shape=jax.ShapeDtypeStruct((B, S, H), x.dtype),
        scratch_shapes=[pltpu.VMEM((T, E), jnp.float32)],
        compiler_params=pltpu.CompilerParams(
            dimension_semantics=("arbitrary", "arbitrary"),
        ),
    )(x, W_router, W_gate, W_gate, W_up, W_up, W_down, W_down)
    return out


# speedup vs baseline: 1.0239x; 1.0085x over previous
import functools

import jax
import jax.numpy as jnp
from jax.experimental import pallas as pl
from jax.experimental.pallas import tpu as pltpu


def _moe_body(x_ref, wr_ref, wg_ref, wu_ref, wd_ref, out_ref, comb_ref):
    e = pl.program_id(0)
    ib = pl.program_id(1)
    T, E = comb_ref.shape

    @pl.when((e == 0) & (ib == 0))
    def _init():
        x = x_ref[...]
        logits = jnp.dot(x, wr_ref[...], preferred_element_type=jnp.float32)
        aff = jax.nn.softmax(logits, axis=-1)
        idx = jax.lax.broadcasted_iota(jnp.int32, (T, E), 1)
        m1 = jnp.max(aff, axis=-1, keepdims=True)
        i1 = jnp.min(jnp.where(aff == m1, idx, E), axis=-1, keepdims=True)
        mask1 = idx == i1
        aff2 = jnp.where(mask1, -1.0, aff)
        m2 = jnp.max(aff2, axis=-1, keepdims=True)
        i2 = jnp.min(jnp.where(aff2 == m2, idx, E), axis=-1, keepdims=True)
        mask2 = idx == i2
        denom = m1 + m2
        comb_ref[...] = (
            jnp.where(mask1, m1 / denom, 0.0) + jnp.where(mask2, m2 / denom, 0.0)
        )
        out_ref[...] = jnp.zeros_like(out_ref)

    x = x_ref[...]
    gate = jnp.dot(x, wg_ref[0], preferred_element_type=jnp.float32)
    up = jnp.dot(x, wu_ref[0], preferred_element_type=jnp.float32)
    hmid = gate * jax.nn.sigmoid(gate) * up

    comb = comb_ref[...]
    idx = jax.lax.broadcasted_iota(jnp.int32, (T, E), 1)
    w_e = jnp.sum(jnp.where(idx == e, comb, 0.0), axis=-1, keepdims=True)
    hmid = hmid * w_e
    out_ref[...] += jnp.dot(hmid, wd_ref[0], preferred_element_type=jnp.float32)


def kernel(x, W_router, W_gate, W_up, W_down):
    B, S, H = x.shape
    E, _, I = W_gate.shape
    T = B * S
    IB = 512
    n_ib = I // IB

    out = pl.pallas_call(
        _moe_body,
        grid=(E, n_ib),
        in_specs=[
            pl.BlockSpec((T, None, H), lambda e, i: (0, 0, 0)),
            pl.BlockSpec((H, E), lambda e, i: (0, 0)),
            pl.BlockSpec((1, H, IB), lambda e, i: (e, 0, i)),
            pl.BlockSpec((1, H, IB), lambda e, i: (e, 0, i)),
            pl.BlockSpec((1, IB, H), lambda e, i: (e, i, 0)),
        ],
        out_specs=pl.BlockSpec((T, None, H), lambda e, i: (0, 0, 0)),
        out_shape=jax.ShapeDtypeStruct((B, S, H), x.dtype),
        scratch_shapes=[pltpu.VMEM((T, E), jnp.float32)],
        compiler_params=pltpu.CompilerParams(
            dimension_semantics=("arbitrary", "arbitrary"),
        ),
    )(x, W_router, W_gate, W_up, W_down)
    return out


# speedup vs baseline: 1.0306x; 1.0065x over previous
import functools

import jax
import jax.numpy as jnp
from jax.experimental import pallas as pl
from jax.experimental.pallas import tpu as pltpu


def _moe_body(x_ref, wr_ref, wg_ref, wu_ref, wd_ref, out_ref, comb_ref):
    e = pl.program_id(1)
    ib = pl.program_id(0)
    T, E = comb_ref.shape

    @pl.when((e == 0) & (ib == 0))
    def _init():
        x = x_ref[...]
        logits = jnp.dot(x, wr_ref[...], preferred_element_type=jnp.float32)
        aff = jax.nn.softmax(logits, axis=-1)
        idx = jax.lax.broadcasted_iota(jnp.int32, (T, E), 1)
        m1 = jnp.max(aff, axis=-1, keepdims=True)
        i1 = jnp.min(jnp.where(aff == m1, idx, E), axis=-1, keepdims=True)
        mask1 = idx == i1
        aff2 = jnp.where(mask1, -1.0, aff)
        m2 = jnp.max(aff2, axis=-1, keepdims=True)
        i2 = jnp.min(jnp.where(aff2 == m2, idx, E), axis=-1, keepdims=True)
        mask2 = idx == i2
        denom = m1 + m2
        comb_ref[...] = (
            jnp.where(mask1, m1 / denom, 0.0) + jnp.where(mask2, m2 / denom, 0.0)
        )
        out_ref[...] = jnp.zeros_like(out_ref)

    x = x_ref[...]
    gate = jnp.dot(x, wg_ref[0], preferred_element_type=jnp.float32)
    up = jnp.dot(x, wu_ref[0], preferred_element_type=jnp.float32)
    hmid = gate * jax.nn.sigmoid(gate) * up

    comb = comb_ref[...]
    idx = jax.lax.broadcasted_iota(jnp.int32, (T, E), 1)
    w_e = jnp.sum(jnp.where(idx == e, comb, 0.0), axis=-1, keepdims=True)
    hmid = hmid * w_e
    out_ref[...] += jnp.dot(hmid, wd_ref[0], preferred_element_type=jnp.float32)


def kernel(x, W_router, W_gate, W_up, W_down):
    B, S, H = x.shape
    E, _, I = W_gate.shape
    T = B * S
    IB = 512
    n_ib = I // IB

    out = pl.pallas_call(
        _moe_body,
        grid=(n_ib, E),
        in_specs=[
            pl.BlockSpec((T, None, H), lambda i, e: (0, 0, 0)),
            pl.BlockSpec((H, E), lambda i, e: (0, 0)),
            pl.BlockSpec((1, H, IB), lambda i, e: (e, 0, i)),
            pl.BlockSpec((1, H, IB), lambda i, e: (e, 0, i)),
            pl.BlockSpec((1, IB, H), lambda i, e: (e, i, 0)),
        ],
        out_specs=pl.BlockSpec((T, None, H), lambda i, e: (0, 0, 0)),
        out_shape=jax.ShapeDtypeStruct((B, S, H), x.dtype),
        scratch_shapes=[pltpu.VMEM((T, E), jnp.float32)],
        compiler_params=pltpu.CompilerParams(
            dimension_semantics=("arbitrary", "arbitrary"),
        ),
    )(x, W_router, W_gate, W_up, W_down)
    return out
